# baseline (speedup 1.0000x reference)
import functools

import jax
import jax.numpy as jnp
from jax import lax
from jax.experimental import pallas as pl
from jax.experimental.pallas import tpu as pltpu
from jax.experimental.pallas import tpu_sc as plsc

_PERIOD = 4096
_ROWS = 16384
_D = 2048
_NC = 2
_NS = 16
_NW = _NC * _NS
_ROWS_PER_W = _ROWS // _NW
_W_PER_GROUP = _PERIOD // _ROWS_PER_W
_B = 8
_NVMEM = 3
_NSH = 4
_NBUF = _NVMEM + _NSH
_RAHEAD = 3
_STEPS = _ROWS_PER_W // _B


@functools.partial(
    pl.kernel,
    mesh=plsc.VectorSubcoreMesh(core_axis_name="c", subcore_axis_name="s"),
    out_type=jax.ShapeDtypeStruct((_ROWS // _PERIOD, _PERIOD, _D), jnp.float32),
    scratch_types=(
        [pltpu.VMEM((_B, _D), jnp.float32) for _ in range(_NVMEM)]
        + [pltpu.VMEM_SHARED((_NS, _NSH, _B, _D), jnp.float32)]
        + [pltpu.SemaphoreType.DMA for _ in range(2 * _NBUF)]
    ),
)
def _gather_view(x_hbm, out_hbm, *scratch):
    vbufs = scratch[:_NVMEM]
    shared = scratch[_NVMEM]
    sems = scratch[_NVMEM + 1:]
    rsems = sems[:_NBUF]
    wsems = sems[_NBUF:]
    c = lax.axis_index("c")
    s = lax.axis_index("s")
    wid = s * _NC + c
    g = wid // _W_PER_GROUP
    off = (wid % _W_PER_GROUP) * _ROWS_PER_W
    base = wid * _ROWS_PER_W

    def buf(i):
        j = i % _NBUF
        if j < _NVMEM:
            return vbufs[j]
        return shared.at[s, j - _NVMEM]

    def read(i):
        return pltpu.make_async_copy(
            x_hbm.at[pl.ds(base + i * _B, _B)], buf(i), rsems[i % _NBUF])

    def write(i):
        return pltpu.make_async_copy(
            buf(i), out_hbm.at[g, pl.ds(off + i * _B, _B)], wsems[i % _NBUF])

    waited = set()
    for i in range(_RAHEAD):
        read(i).start()
    for i in range(_STEPS):
        read(i).wait()
        write(i).start()
        nxt = i + _RAHEAD
        if nxt < _STEPS:
            j = nxt - _NBUF
            if j >= 0:
                write(j).wait()
                waited.add(j)
            read(nxt).start()
    for i in range(_STEPS):
        if i not in waited:
            write(i).wait()


def kernel(x):
    return _gather_view(x)


# speedup vs baseline: 1.0337x; 1.0337x over previous
import functools

import jax
import jax.numpy as jnp
from jax import lax
from jax.experimental import pallas as pl
from jax.experimental.pallas import tpu as pltpu
from jax.experimental.pallas import tpu_sc as plsc

_PERIOD = 4096
_ROWS = 16384
_D = 2048
_NC = 2
_NS = 16
_NW = _NC * _NS
_ROWS_PER_W = _ROWS // _NW
_W_PER_GROUP = _PERIOD // _ROWS_PER_W
_B = 8
_NBUF = 4
_STEPS = _ROWS_PER_W // _B
_NGROUPS = (_STEPS - _NBUF) // _NBUF


@functools.partial(
    pl.kernel,
    mesh=plsc.VectorSubcoreMesh(core_axis_name="c", subcore_axis_name="s"),
    out_type=jax.ShapeDtypeStruct((_ROWS // _PERIOD, _PERIOD, _D), jnp.float32),
    scratch_types=(
        [pltpu.VMEM_SHARED((_NS, _NBUF, _B, _D), jnp.float32)]
        + [pltpu.SemaphoreType.DMA for _ in range(2 * _NBUF)]
    ),
)
def _gather_view(x_hbm, out_hbm, shared, *sems):
    rsems = sems[:_NBUF]
    wsems = sems[_NBUF:]
    s = lax.axis_index("s")
    wid = s * _NC + lax.axis_index("c")
    g = wid // _W_PER_GROUP
    off = (wid % _W_PER_GROUP) * _ROWS_PER_W
    base = wid * _ROWS_PER_W

    def read(i, b):
        return pltpu.make_async_copy(
            x_hbm.at[pl.ds(base + i * _B, _B)], shared.at[s, b], rsems[b])

    def write(i, b):
        return pltpu.make_async_copy(
            shared.at[s, b], out_hbm.at[g, pl.ds(off + i * _B, _B)], wsems[b])

    for b in range(_NBUF):
        read(b, b).start()

    @pl.loop(0, _NGROUPS)
    def _loop(t):
        i0 = t * _NBUF
        for b in range(_NBUF):
            i = i0 + b
            read(i, b).wait()
            write(i, b).start()
            write(i, b).wait()
            read(i + _NBUF, b).start()

    for b in range(_NBUF):
        i = _NGROUPS * _NBUF + b
        read(i, b).wait()
        write(i, b).start()
        write(i, b).wait()


def kernel(x):
    return _gather_view(x)


# speedup vs baseline: 1.0477x; 1.0136x over previous
import functools

import jax
import jax.numpy as jnp
from jax import lax
from jax.experimental import pallas as pl
from jax.experimental.pallas import tpu as pltpu
from jax.experimental.pallas import tpu_sc as plsc

_PERIOD = 4096
_ROWS = 16384
_D = 2048
_NC = 2
_NS = 16
_NW = _NC * _NS
_ROWS_PER_W = _ROWS // _NW
_W_PER_GROUP = _PERIOD // _ROWS_PER_W
_B = 8
_NBUF = 4
_STEPS = _ROWS_PER_W // _B
_NGRP = _STEPS // _NBUF


@functools.partial(
    pl.kernel,
    mesh=plsc.VectorSubcoreMesh(core_axis_name="c", subcore_axis_name="s"),
    out_type=jax.ShapeDtypeStruct((_ROWS // _PERIOD, _PERIOD, _D), jnp.float32),
    scratch_types=(
        [pltpu.VMEM_SHARED((_NS, _NBUF, _B, _D), jnp.float32)]
        + [pltpu.SemaphoreType.DMA for _ in range(2 * _NBUF)]
    ),
)
def _gather_view(x_hbm, out_hbm, shared, *sems):
    rsems = sems[:_NBUF]
    wsems = sems[_NBUF:]
    s = lax.axis_index("s")
    wid = s * _NC + lax.axis_index("c")
    g = wid // _W_PER_GROUP
    off = (wid % _W_PER_GROUP) * _ROWS_PER_W
    base = wid * _ROWS_PER_W

    def read(i, b):
        return pltpu.make_async_copy(
            x_hbm.at[pl.ds(base + i * _B, _B)], shared.at[s, b], rsems[b])

    def write(i, b):
        return pltpu.make_async_copy(
            shared.at[s, b], out_hbm.at[g, pl.ds(off + i * _B, _B)], wsems[b])

    read(0, 0).start()
    read(1, 1).start()

    read(0, 0).wait(); write(0, 0).start(); read(2, 2).start()
    read(1, 1).wait(); write(1, 1).start(); read(3, 3).start()
    read(2, 2).wait(); write(2, 2).start(); write(0, 0).wait(); read(4, 0).start()
    read(3, 3).wait(); write(3, 3).start(); write(1, 1).wait(); read(5, 1).start()

    @pl.loop(1, _NGRP - 1)
    def _loop(t):
        i0 = t * _NBUF
        for b in range(_NBUF):
            i = i0 + b
            read(i, b).wait()
            write(i, b).start()
            write(i, (b + 2) % _NBUF).wait()
            read(i + 2, (b + 2) % _NBUF).start()

    i0 = (_NGRP - 1) * _NBUF
    read(i0 + 0, 0).wait(); write(i0 + 0, 0).start(); write(i0 - 2, 2).wait(); read(i0 + 2, 2).start()
    read(i0 + 1, 1).wait(); write(i0 + 1, 1).start(); write(i0 - 1, 3).wait(); read(i0 + 3, 3).start()
    read(i0 + 2, 2).wait(); write(i0 + 2, 2).start(); write(i0 + 0, 0).wait()
    read(i0 + 3, 3).wait(); write(i0 + 3, 3).start(); write(i0 + 1, 1).wait()
    write(i0 + 2, 2).wait()
    write(i0 + 3, 3).wait()


def kernel(x):
    return _gather_view(x)
